# baseline (speedup 1.0000x reference)
import functools

import jax
import jax.numpy as jnp
from jax import lax
from jax.experimental import pallas as pl
from jax.experimental.pallas import tpu as pltpu
from jax.experimental.pallas import tpu_sc as plsc

_VOCAB = 1000000
_D = 64
_B = 16384
_SEQ = 200

_NC = 2
_NS = 16
_NW = _NC * _NS
_BPW = _B // _NW
_CHUNK = 128
_NCHUNK = _BPW // _CHUNK
_JC = 40
_NJ = _SEQ // _JC


def _sc_pool(text, emb):
  mesh = plsc.VectorSubcoreMesh(core_axis_name="c", subcore_axis_name="s")

  @functools.partial(
      pl.kernel,
      out_type=(
          jax.ShapeDtypeStruct((_B, _D), jnp.float32),
          jax.ShapeDtypeStruct((_B,), jnp.float32),
      ),
      mesh=mesh,
      scratch_types=[
          pltpu.VMEM((_BPW, _JC), jnp.int32),
          pltpu.VMEM((_JC, _NCHUNK, _CHUNK), jnp.int32),
          pltpu.VMEM((_BPW, _D), jnp.float32),
          pltpu.VMEM((_BPW,), jnp.float32),
          pltpu.SemaphoreType.DMA,
      ],
      compiler_params=pltpu.CompilerParams(
          use_tc_tiling_on_sc=False, needs_layout_passes=False),
  )
  def pool(text_hbm, emb_hbm, out_hbm, cnt_hbm, blk_v, idxT_v, acc_v, cnt_v,
           sem):
    wid = lax.axis_index("s") * _NC + lax.axis_index("c")
    base = wid * _BPW
    iota = lax.broadcasted_iota(jnp.int32, (16,), 0)

    zero16 = jnp.zeros((16,), jnp.float32)
    for z in range(_BPW // 16):
      cnt_v[pl.ds(z * 16, 16)] = zero16

    for h in range(_NJ):
      pltpu.sync_copy(
          text_hbm.at[pl.ds(base, _BPW), pl.ds(h * _JC, _JC)], blk_v)

      @pl.loop(0, _JC)
      def _(j):
        jcol = jnp.full((16,), j, jnp.int32)
        for c in range(_NCHUNK):
          for lb in range(_CHUNK // 16):
            row0 = c * _CHUNK + lb * 16
            vals = plsc.load_gather(blk_v, [iota + row0, jcol])
            idxT_v[j, c, pl.ds(lb * 16, 16)] = vals
            cnt_v[pl.ds(row0, 16)] = cnt_v[pl.ds(row0, 16)] + jnp.where(
                vals == 0, 1.0, 0.0)

      if h == 0:
        for c in range(_NCHUNK):
          pltpu.async_copy(emb_hbm.at[idxT_v.at[0, c]],
                           acc_v.at[pl.ds(c * _CHUNK, _CHUNK)], sem)
        for c in range(_NCHUNK):
          pltpu.make_async_copy(emb_hbm.at[idxT_v.at[0, c]],
                                acc_v.at[pl.ds(c * _CHUNK, _CHUNK)],
                                sem).wait()

      @pl.loop(1 if h == 0 else 0, _JC)
      def _(j):
        for c in range(_NCHUNK):
          pltpu.async_copy(emb_hbm.at[idxT_v.at[j, c]],
                           acc_v.at[pl.ds(c * _CHUNK, _CHUNK)], sem, add=True)
        for c in range(_NCHUNK):
          pltpu.make_async_copy(emb_hbm.at[idxT_v.at[j, c]],
                                acc_v.at[pl.ds(c * _CHUNK, _CHUNK)],
                                sem).wait()

    pltpu.sync_copy(acc_v, out_hbm.at[pl.ds(base, _BPW)])
    pltpu.sync_copy(cnt_v, cnt_hbm.at[pl.ds(base, _BPW)])

  return pool(text, emb)


def _mlp_block(acc_ref, cnt_ref, emb0_ref, w1_ref, b1_ref, w2_ref, b2_ref,
               wo_ref, bo_ref, out_ref):
  pooled = (acc_ref[...] - cnt_ref[...] * emb0_ref[...]) * (1.0 / _SEQ)
  h = jnp.dot(pooled, w1_ref[...], preferred_element_type=jnp.float32)
  h = jnp.maximum(h + b1_ref[...], 0.0)
  h = jnp.dot(h, w2_ref[...], preferred_element_type=jnp.float32)
  h = jnp.maximum(h + b2_ref[...], 0.0)
  out_ref[...] = (
      jnp.dot(h, wo_ref[...], preferred_element_type=jnp.float32)
      + bo_ref[...])


def _tc_mlp(acc, cnt, emb0, W1, b1, W2, b2, Wo, bo):
  bblk = 2048
  grid = (_B // bblk,)
  full = lambda shape: pl.BlockSpec(shape, lambda i: (0, 0))
  return pl.pallas_call(
      _mlp_block,
      grid=grid,
      in_specs=[
          pl.BlockSpec((bblk, _D), lambda i: (i, 0)),
          pl.BlockSpec((bblk, 1), lambda i: (i, 0)),
          full((1, _D)),
          full(W1.shape),
          full((1, 256)),
          full(W2.shape),
          full((1, 128)),
          full(Wo.shape),
          full((1, 1)),
      ],
      out_specs=pl.BlockSpec((bblk, 1), lambda i: (i, 0)),
      out_shape=jax.ShapeDtypeStruct((_B, 1), jnp.float32),
  )(acc, cnt, emb0, W1, b1.reshape(1, -1), W2, b2.reshape(1, -1), Wo,
    bo.reshape(1, -1))


def kernel(text, emb, W1, b1, W2, b2, Wo, bo):
  acc, cnt = _sc_pool(text, emb)
  return _tc_mlp(acc, cnt.reshape(_B, 1), emb[0:1], W1, b1, W2, b2, Wo, bo)


# speedup vs baseline: 1.1177x; 1.1177x over previous
import functools

import jax
import jax.numpy as jnp
from jax import lax
from jax.experimental import pallas as pl
from jax.experimental.pallas import tpu as pltpu
from jax.experimental.pallas import tpu_sc as plsc

_VOCAB = 1000000
_D = 64
_B = 16384
_SEQ = 200

_NC = 2
_NS = 16
_NW = _NC * _NS
_BPW = _B // _NW
_CHUNK = 128
_NCHUNK = _BPW // _CHUNK
_JC = 40
_NJ = _SEQ // _JC


def _pad_block(inT_ref, out_ref):
  out_ref[:, 0:_D] = inT_ref[...].T
  out_ref[:, _D:2 * _D] = jnp.zeros((out_ref.shape[0], _D), jnp.float32)


def _tc_padT(embT):
  vb = 2048
  grid = (pl.cdiv(_VOCAB, vb),)
  return pl.pallas_call(
      _pad_block,
      grid=grid,
      in_specs=[pl.BlockSpec((_D, vb), lambda i: (0, i))],
      out_specs=pl.BlockSpec((vb, 2 * _D), lambda i: (i, 0)),
      out_shape=jax.ShapeDtypeStruct((_VOCAB, 2 * _D), jnp.float32),
  )(embT)


def _sc_pool(text, emb2):
  mesh = plsc.VectorSubcoreMesh(core_axis_name="c", subcore_axis_name="s")

  @functools.partial(
      pl.kernel,
      out_type=(
          jax.ShapeDtypeStruct((_B, _D), jnp.float32),
          jax.ShapeDtypeStruct((_B,), jnp.float32),
      ),
      mesh=mesh,
      scratch_types=[
          pltpu.VMEM((_BPW, _JC), jnp.int32),
          pltpu.VMEM((_JC, _NCHUNK, _CHUNK), jnp.int32),
          pltpu.VMEM((_BPW, _D), jnp.float32),
          pltpu.VMEM((_BPW,), jnp.float32),
          pltpu.SemaphoreType.DMA,
      ],
      compiler_params=pltpu.CompilerParams(
          use_tc_tiling_on_sc=False, needs_layout_passes=False),
  )
  def pool(text_hbm, emb_hbm, out_hbm, cnt_hbm, blk_v, idxT_v, acc_v, cnt_v,
           sem):
    wid = lax.axis_index("s") * _NC + lax.axis_index("c")
    base = wid * _BPW
    iota = lax.broadcasted_iota(jnp.int32, (16,), 0)

    zero16 = jnp.zeros((16,), jnp.float32)
    for z in range(_BPW // 16):
      cnt_v[pl.ds(z * 16, 16)] = zero16

    for h in range(_NJ):
      pltpu.sync_copy(
          text_hbm.at[pl.ds(base, _BPW), pl.ds(h * _JC, _JC)], blk_v)

      @pl.loop(0, _JC)
      def _(j):
        jcol = jnp.full((16,), j, jnp.int32)
        for c in range(_NCHUNK):
          for lb in range(_CHUNK // 16):
            row0 = c * _CHUNK + lb * 16
            vals = plsc.load_gather(blk_v, [iota + row0, jcol])
            idxT_v[j, c, pl.ds(lb * 16, 16)] = vals + vals
            cnt_v[pl.ds(row0, 16)] = cnt_v[pl.ds(row0, 16)] + jnp.where(
                vals == 0, 1.0, 0.0)

      if h == 0:
        for c in range(_NCHUNK):
          pltpu.async_copy(emb_hbm.at[idxT_v.at[0, c]],
                           acc_v.at[pl.ds(c * _CHUNK, _CHUNK)], sem)
        for c in range(_NCHUNK):
          pltpu.make_async_copy(emb_hbm.at[idxT_v.at[0, c]],
                                acc_v.at[pl.ds(c * _CHUNK, _CHUNK)],
                                sem).wait()

      @pl.loop(1 if h == 0 else 0, _JC)
      def _(j):
        for c in range(_NCHUNK):
          pltpu.async_copy(emb_hbm.at[idxT_v.at[j, c]],
                           acc_v.at[pl.ds(c * _CHUNK, _CHUNK)], sem, add=True)
        for c in range(_NCHUNK):
          pltpu.make_async_copy(emb_hbm.at[idxT_v.at[j, c]],
                                acc_v.at[pl.ds(c * _CHUNK, _CHUNK)],
                                sem).wait()

    pltpu.sync_copy(acc_v, out_hbm.at[pl.ds(base, _BPW)])
    pltpu.sync_copy(cnt_v, cnt_hbm.at[pl.ds(base, _BPW)])

  return pool(text, emb2)


def _mlp_block(acc_ref, cnt_ref, emb0_ref, w1_ref, b1_ref, w2_ref, b2_ref,
               wo_ref, bo_ref, out_ref):
  pooled = (acc_ref[...] - cnt_ref[...] * emb0_ref[...]) * (1.0 / _SEQ)
  h = jnp.dot(pooled, w1_ref[...], preferred_element_type=jnp.float32)
  h = jnp.maximum(h + b1_ref[...], 0.0)
  h = jnp.dot(h, w2_ref[...], preferred_element_type=jnp.float32)
  h = jnp.maximum(h + b2_ref[...], 0.0)
  out_ref[...] = (
      jnp.dot(h, wo_ref[...], preferred_element_type=jnp.float32)
      + bo_ref[...])


def _tc_mlp(acc, cnt, emb0, W1, b1, W2, b2, Wo, bo):
  bblk = 2048
  grid = (_B // bblk,)
  full = lambda shape: pl.BlockSpec(shape, lambda i: (0, 0))
  return pl.pallas_call(
      _mlp_block,
      grid=grid,
      in_specs=[
          pl.BlockSpec((bblk, _D), lambda i: (i, 0)),
          pl.BlockSpec((bblk, 1), lambda i: (i, 0)),
          full((1, _D)),
          full(W1.shape),
          full((1, 256)),
          full(W2.shape),
          full((1, 128)),
          full(Wo.shape),
          full((1, 1)),
      ],
      out_specs=pl.BlockSpec((bblk, 1), lambda i: (i, 0)),
      out_shape=jax.ShapeDtypeStruct((_B, 1), jnp.float32),
  )(acc, cnt, emb0, W1, b1.reshape(1, -1), W2, b2.reshape(1, -1), Wo,
    bo.reshape(1, -1))


def kernel(text, emb, W1, b1, W2, b2, Wo, bo):
  emb2 = _tc_padT(emb.T).reshape(2 * _VOCAB, _D)
  acc, cnt = _sc_pool(text, emb2)
  return _tc_mlp(acc, cnt.reshape(_B, 1), emb[0:1], W1, b1, W2, b2, Wo, bo)


# speedup vs baseline: 1.1184x; 1.0006x over previous
import functools

import jax
import jax.numpy as jnp
from jax import lax
from jax.experimental import pallas as pl
from jax.experimental.pallas import tpu as pltpu
from jax.experimental.pallas import tpu_sc as plsc

_VOCAB = 1000000
_D = 64
_B = 16384
_SEQ = 200

_NC = 2
_NS = 16
_NW = _NC * _NS
_BPW = _B // _NW
_CHUNK = 128
_NCHUNK = _BPW // _CHUNK
_JC = 40
_NJ = _SEQ // _JC


def _pad_block(inT_ref, out_ref):
  out_ref[:, 0:_D] = inT_ref[...].T


def _tc_padT(embT):
  vb = 2048
  grid = (pl.cdiv(_VOCAB, vb),)
  return pl.pallas_call(
      _pad_block,
      grid=grid,
      in_specs=[pl.BlockSpec((_D, vb), lambda i: (0, i))],
      out_specs=pl.BlockSpec((vb, 2 * _D), lambda i: (i, 0)),
      out_shape=jax.ShapeDtypeStruct((_VOCAB, 2 * _D), jnp.float32),
  )(embT)


def _sc_pool(text, emb2):
  mesh = plsc.VectorSubcoreMesh(core_axis_name="c", subcore_axis_name="s")

  @functools.partial(
      pl.kernel,
      out_type=(
          jax.ShapeDtypeStruct((_B, _D), jnp.float32),
          jax.ShapeDtypeStruct((_B,), jnp.float32),
      ),
      mesh=mesh,
      scratch_types=[
          pltpu.VMEM((_BPW, _JC), jnp.int32),
          pltpu.VMEM((_JC, _NCHUNK, _CHUNK), jnp.int32),
          pltpu.VMEM((_BPW, _D), jnp.float32),
          pltpu.VMEM((_BPW,), jnp.float32),
          pltpu.SemaphoreType.DMA,
      ],
      compiler_params=pltpu.CompilerParams(
          use_tc_tiling_on_sc=False, needs_layout_passes=False),
  )
  def pool(text_hbm, emb_hbm, out_hbm, cnt_hbm, blk_v, idxT_v, acc_v, cnt_v,
           sem):
    wid = lax.axis_index("s") * _NC + lax.axis_index("c")
    base = wid * _BPW
    iota = lax.broadcasted_iota(jnp.int32, (16,), 0)

    zero16 = jnp.zeros((16,), jnp.float32)
    for z in range(_BPW // 16):
      cnt_v[pl.ds(z * 16, 16)] = zero16

    for h in range(_NJ):
      pltpu.sync_copy(
          text_hbm.at[pl.ds(base, _BPW), pl.ds(h * _JC, _JC)], blk_v)

      @pl.loop(0, _JC)
      def _(j):
        jcol = jnp.full((16,), j, jnp.int32)
        for c in range(_NCHUNK):
          for lb in range(_CHUNK // 16):
            row0 = c * _CHUNK + lb * 16
            vals = plsc.load_gather(blk_v, [iota + row0, jcol])
            idxT_v[j, c, pl.ds(lb * 16, 16)] = vals + vals
            cnt_v[pl.ds(row0, 16)] = cnt_v[pl.ds(row0, 16)] + jnp.where(
                vals == 0, 1.0, 0.0)

      if h == 0:
        for c in range(_NCHUNK):
          pltpu.async_copy(emb_hbm.at[idxT_v.at[0, c]],
                           acc_v.at[pl.ds(c * _CHUNK, _CHUNK)], sem)
        for c in range(_NCHUNK):
          pltpu.make_async_copy(emb_hbm.at[idxT_v.at[0, c]],
                                acc_v.at[pl.ds(c * _CHUNK, _CHUNK)],
                                sem).wait()

      @pl.loop(1 if h == 0 else 0, _JC)
      def _(j):
        for c in range(_NCHUNK):
          pltpu.async_copy(emb_hbm.at[idxT_v.at[j, c]],
                           acc_v.at[pl.ds(c * _CHUNK, _CHUNK)], sem, add=True)
        for c in range(_NCHUNK):
          pltpu.make_async_copy(emb_hbm.at[idxT_v.at[j, c]],
                                acc_v.at[pl.ds(c * _CHUNK, _CHUNK)],
                                sem).wait()

    pltpu.sync_copy(acc_v, out_hbm.at[pl.ds(base, _BPW)])
    pltpu.sync_copy(cnt_v, cnt_hbm.at[pl.ds(base, _BPW)])

  return pool(text, emb2)


def _mlp_block(acc_ref, cnt_ref, emb0_ref, w1_ref, b1_ref, w2_ref, b2_ref,
               wo_ref, bo_ref, out_ref):
  pooled = (acc_ref[...] - cnt_ref[...] * emb0_ref[...]) * (1.0 / _SEQ)
  h = jnp.dot(pooled, w1_ref[...], preferred_element_type=jnp.float32)
  h = jnp.maximum(h + b1_ref[...], 0.0)
  h = jnp.dot(h, w2_ref[...], preferred_element_type=jnp.float32)
  h = jnp.maximum(h + b2_ref[...], 0.0)
  out_ref[...] = (
      jnp.dot(h, wo_ref[...], preferred_element_type=jnp.float32)
      + bo_ref[...])


def _tc_mlp(acc, cnt, emb0, W1, b1, W2, b2, Wo, bo):
  bblk = 2048
  grid = (_B // bblk,)
  full = lambda shape: pl.BlockSpec(shape, lambda i: (0, 0))
  return pl.pallas_call(
      _mlp_block,
      grid=grid,
      in_specs=[
          pl.BlockSpec((bblk, _D), lambda i: (i, 0)),
          pl.BlockSpec((bblk, 1), lambda i: (i, 0)),
          full((1, _D)),
          full(W1.shape),
          full((1, 256)),
          full(W2.shape),
          full((1, 128)),
          full(Wo.shape),
          full((1, 1)),
      ],
      out_specs=pl.BlockSpec((bblk, 1), lambda i: (i, 0)),
      out_shape=jax.ShapeDtypeStruct((_B, 1), jnp.float32),
  )(acc, cnt, emb0, W1, b1.reshape(1, -1), W2, b2.reshape(1, -1), Wo,
    bo.reshape(1, -1))


def kernel(text, emb, W1, b1, W2, b2, Wo, bo):
  emb2 = _tc_padT(emb.T).reshape(2 * _VOCAB, _D)
  acc, cnt = _sc_pool(text, emb2)
  return _tc_mlp(acc, cnt.reshape(_B, 1), emb[0:1], W1, b1, W2, b2, Wo, bo)


# speedup vs baseline: 1.2853x; 1.1493x over previous
import functools

import jax
import jax.numpy as jnp
from jax import lax
from jax.experimental import pallas as pl
from jax.experimental.pallas import tpu as pltpu
from jax.experimental.pallas import tpu_sc as plsc

_VOCAB = 1000000
_D = 64
_B = 16384
_SEQ = 200

_NC = 2
_NS = 16
_NW = _NC * _NS
_BPW = _B // _NW
_CHUNK = 128
_NCHUNK = _BPW // _CHUNK
_JC = 40
_NJ = _SEQ // _JC


_S = 501760
_VBT = 2048


def _pack_block(a_ref, b_ref, out_ref):
  out_ref[:, 0:_D] = a_ref[...].T
  out_ref[:, _D:2 * _D] = b_ref[...].T


def _tc_packT(embT):
  grid = (_S // _VBT,)
  return pl.pallas_call(
      _pack_block,
      grid=grid,
      in_specs=[
          pl.BlockSpec((_D, _VBT), lambda i: (0, i)),
          pl.BlockSpec(
              (_D, _VBT),
              lambda i: (0, jnp.minimum(i + _S // _VBT,
                                        pl.cdiv(_VOCAB, _VBT) - 1))),
      ],
      out_specs=pl.BlockSpec((_VBT, 2 * _D), lambda i: (i, 0)),
      out_shape=jax.ShapeDtypeStruct((_S, 2 * _D), jnp.float32),
  )(embT, embT)


def _sc_pool(text, emb2):
  mesh = plsc.VectorSubcoreMesh(core_axis_name="c", subcore_axis_name="s")

  @functools.partial(
      pl.kernel,
      out_type=(
          jax.ShapeDtypeStruct((_B, _D), jnp.float32),
          jax.ShapeDtypeStruct((_B,), jnp.float32),
      ),
      mesh=mesh,
      scratch_types=[
          pltpu.VMEM((_BPW, _JC), jnp.int32),
          pltpu.VMEM((_JC, _NCHUNK, _CHUNK), jnp.int32),
          pltpu.VMEM((_BPW, _D), jnp.float32),
          pltpu.VMEM((_BPW,), jnp.float32),
          pltpu.SemaphoreType.DMA,
      ],
      compiler_params=pltpu.CompilerParams(
          use_tc_tiling_on_sc=False, needs_layout_passes=False),
  )
  def pool(text_hbm, emb_hbm, out_hbm, cnt_hbm, blk_v, idxT_v, acc_v, cnt_v,
           sem):
    wid = lax.axis_index("s") * _NC + lax.axis_index("c")
    base = wid * _BPW
    iota = lax.broadcasted_iota(jnp.int32, (16,), 0)

    zero16 = jnp.zeros((16,), jnp.float32)
    for z in range(_BPW // 16):
      cnt_v[pl.ds(z * 16, 16)] = zero16

    for h in range(_NJ):
      pltpu.sync_copy(
          text_hbm.at[pl.ds(base, _BPW), pl.ds(h * _JC, _JC)], blk_v)

      @pl.loop(0, _JC)
      def _(j):
        jcol = jnp.full((16,), j, jnp.int32)
        for c in range(_NCHUNK):
          for lb in range(_CHUNK // 16):
            row0 = c * _CHUNK + lb * 16
            vals = plsc.load_gather(blk_v, [iota + row0, jcol])
            vals2 = vals + vals
            idxT_v[j, c, pl.ds(lb * 16, 16)] = jnp.where(
                vals < _S, vals2, vals2 - (2 * _S - 1))
            cnt_v[pl.ds(row0, 16)] = cnt_v[pl.ds(row0, 16)] + jnp.where(
                vals == 0, 1.0, 0.0)

      if h == 0:
        for c in range(_NCHUNK):
          pltpu.async_copy(emb_hbm.at[idxT_v.at[0, c]],
                           acc_v.at[pl.ds(c * _CHUNK, _CHUNK)], sem)
        for c in range(_NCHUNK):
          pltpu.make_async_copy(emb_hbm.at[idxT_v.at[0, c]],
                                acc_v.at[pl.ds(c * _CHUNK, _CHUNK)],
                                sem).wait()

      @pl.loop(1 if h == 0 else 0, _JC)
      def _(j):
        for c in range(_NCHUNK):
          pltpu.async_copy(emb_hbm.at[idxT_v.at[j, c]],
                           acc_v.at[pl.ds(c * _CHUNK, _CHUNK)], sem, add=True)
        for c in range(_NCHUNK):
          pltpu.make_async_copy(emb_hbm.at[idxT_v.at[j, c]],
                                acc_v.at[pl.ds(c * _CHUNK, _CHUNK)],
                                sem).wait()

    pltpu.sync_copy(acc_v, out_hbm.at[pl.ds(base, _BPW)])
    pltpu.sync_copy(cnt_v, cnt_hbm.at[pl.ds(base, _BPW)])

  return pool(text, emb2)


def _mlp_block(acc_ref, cnt_ref, emb0_ref, w1_ref, b1_ref, w2_ref, b2_ref,
               wo_ref, bo_ref, out_ref):
  pooled = (acc_ref[...] - cnt_ref[...] * emb0_ref[...]) * (1.0 / _SEQ)
  h = jnp.dot(pooled, w1_ref[...], preferred_element_type=jnp.float32)
  h = jnp.maximum(h + b1_ref[...], 0.0)
  h = jnp.dot(h, w2_ref[...], preferred_element_type=jnp.float32)
  h = jnp.maximum(h + b2_ref[...], 0.0)
  out_ref[...] = (
      jnp.dot(h, wo_ref[...], preferred_element_type=jnp.float32)
      + bo_ref[...])


def _tc_mlp(acc, cnt, emb0, W1, b1, W2, b2, Wo, bo):
  bblk = 2048
  grid = (_B // bblk,)
  full = lambda shape: pl.BlockSpec(shape, lambda i: (0, 0))
  return pl.pallas_call(
      _mlp_block,
      grid=grid,
      in_specs=[
          pl.BlockSpec((bblk, _D), lambda i: (i, 0)),
          pl.BlockSpec((bblk, 1), lambda i: (i, 0)),
          full((1, _D)),
          full(W1.shape),
          full((1, 256)),
          full(W2.shape),
          full((1, 128)),
          full(Wo.shape),
          full((1, 1)),
      ],
      out_specs=pl.BlockSpec((bblk, 1), lambda i: (i, 0)),
      out_shape=jax.ShapeDtypeStruct((_B, 1), jnp.float32),
  )(acc, cnt, emb0, W1, b1.reshape(1, -1), W2, b2.reshape(1, -1), Wo,
    bo.reshape(1, -1))


def kernel(text, emb, W1, b1, W2, b2, Wo, bo):
  emb2 = _tc_packT(emb.T).reshape(2 * _S, _D)
  acc, cnt = _sc_pool(text, emb2)
  return _tc_mlp(acc, cnt.reshape(_B, 1), emb[0:1], W1, b1, W2, b2, Wo, bo)


# speedup vs baseline: 1.5139x; 1.1779x over previous
import functools

import jax
import jax.numpy as jnp
from jax import lax
from jax.experimental import pallas as pl
from jax.experimental.pallas import tpu as pltpu
from jax.experimental.pallas import tpu_sc as plsc

_VOCAB = 1000000
_D = 64
_B = 16384
_SEQ = 200

_NC = 2
_NS = 16
_NW = _NC * _NS
_BPW = _B // _NW
_CHUNK = 128
_NCHUNK = _BPW // _CHUNK
_JC = 40
_NJ = _SEQ // _JC


_S = 501760
_VBT = 2048


def _pack_block(a_ref, b_ref, out_ref):
  out_ref[:, 0:_D] = a_ref[...].T
  out_ref[:, _D:2 * _D] = b_ref[...].T


def _tc_packT(embT):
  grid = (_S // _VBT,)
  return pl.pallas_call(
      _pack_block,
      grid=grid,
      in_specs=[
          pl.BlockSpec((_D, _VBT), lambda i: (0, i)),
          pl.BlockSpec(
              (_D, _VBT),
              lambda i: (0, jnp.minimum(i + _S // _VBT,
                                        pl.cdiv(_VOCAB, _VBT) - 1))),
      ],
      out_specs=pl.BlockSpec((_VBT, 2 * _D), lambda i: (i, 0)),
      out_shape=jax.ShapeDtypeStruct((_S, 2 * _D), jnp.float32),
  )(embT, embT)


def _sc_pool(text, emb2):
  mesh = plsc.VectorSubcoreMesh(core_axis_name="c", subcore_axis_name="s")

  @functools.partial(
      pl.kernel,
      out_type=(
          jax.ShapeDtypeStruct((_B, _D), jnp.float32),
          jax.ShapeDtypeStruct((_B,), jnp.float32),
      ),
      mesh=mesh,
      scratch_types=[
          pltpu.VMEM((_BPW, _JC), jnp.int32),
          pltpu.VMEM((_JC, _NCHUNK, _CHUNK), jnp.int32),
          pltpu.VMEM((_JC, _NCHUNK, _CHUNK), jnp.int32),
          pltpu.VMEM((_BPW, _D), jnp.float32),
          pltpu.VMEM((_BPW,), jnp.float32),
          pltpu.SemaphoreType.DMA,
      ],
      compiler_params=pltpu.CompilerParams(
          use_tc_tiling_on_sc=False, needs_layout_passes=False),
  )
  def pool(text_hbm, emb_hbm, out_hbm, cnt_hbm, blk_v, idxTa_v, idxTb_v,
           acc_v, cnt_v, sem):
    wid = lax.axis_index("s") * _NC + lax.axis_index("c")
    base = wid * _BPW
    iota = lax.broadcasted_iota(jnp.int32, (16,), 0)
    idx_bufs = (idxTa_v, idxTb_v)

    zero16 = jnp.zeros((16,), jnp.float32)
    for z in range(_BPW // 16):
      cnt_v[pl.ds(z * 16, 16)] = zero16

    @pl.loop(0, _BPW)
    def _(r):
      for k in range(_D // 16):
        acc_v[r, pl.ds(k * 16, 16)] = zero16

    def load_and_transpose(h, idxT_v):
      pltpu.sync_copy(
          text_hbm.at[pl.ds(base, _BPW), pl.ds(h * _JC, _JC)], blk_v)

      @pl.loop(0, _JC)
      def _(j):
        jcol = jnp.full((16,), j, jnp.int32)
        for c in range(_NCHUNK):
          for lb in range(_CHUNK // 16):
            row0 = c * _CHUNK + lb * 16
            vals = plsc.load_gather(blk_v, [iota + row0, jcol])
            vals2 = vals + vals
            idxT_v[j, c, pl.ds(lb * 16, 16)] = jnp.where(
                vals < _S, vals2, vals2 - (2 * _S - 1))
            cnt_v[pl.ds(row0, 16)] = cnt_v[pl.ds(row0, 16)] + jnp.where(
                vals == 0, 1.0, 0.0)

    load_and_transpose(0, idxTa_v)
    for h in range(_NJ):
      idxT_v = idx_bufs[h % 2]

      @pl.loop(0, _JC)
      def _(j):
        for c in range(_NCHUNK):
          pltpu.async_copy(emb_hbm.at[idxT_v.at[j, c]],
                           acc_v.at[pl.ds(c * _CHUNK, _CHUNK)], sem, add=True)

      if h + 1 < _NJ:
        load_and_transpose(h + 1, idx_bufs[(h + 1) % 2])

      @pl.loop(0, _JC * _NCHUNK)
      def _(i):
        pltpu.make_async_copy(emb_hbm.at[idxT_v.at[0, 0]],
                              acc_v.at[pl.ds(0, _CHUNK)], sem).wait()

    pltpu.sync_copy(acc_v, out_hbm.at[pl.ds(base, _BPW)])
    pltpu.sync_copy(cnt_v, cnt_hbm.at[pl.ds(base, _BPW)])

  return pool(text, emb2)


def _mlp_block(acc_ref, cnt_ref, emb0_ref, w1_ref, b1_ref, w2_ref, b2_ref,
               wo_ref, bo_ref, out_ref):
  pooled = (acc_ref[...] - cnt_ref[...] * emb0_ref[...]) * (1.0 / _SEQ)
  h = jnp.dot(pooled, w1_ref[...], preferred_element_type=jnp.float32)
  h = jnp.maximum(h + b1_ref[...], 0.0)
  h = jnp.dot(h, w2_ref[...], preferred_element_type=jnp.float32)
  h = jnp.maximum(h + b2_ref[...], 0.0)
  out_ref[...] = (
      jnp.dot(h, wo_ref[...], preferred_element_type=jnp.float32)
      + bo_ref[...])


def _tc_mlp(acc, cnt, emb0, W1, b1, W2, b2, Wo, bo):
  bblk = 2048
  grid = (_B // bblk,)
  full = lambda shape: pl.BlockSpec(shape, lambda i: (0, 0))
  return pl.pallas_call(
      _mlp_block,
      grid=grid,
      in_specs=[
          pl.BlockSpec((bblk, _D), lambda i: (i, 0)),
          pl.BlockSpec((bblk, 1), lambda i: (i, 0)),
          full((1, _D)),
          full(W1.shape),
          full((1, 256)),
          full(W2.shape),
          full((1, 128)),
          full(Wo.shape),
          full((1, 1)),
      ],
      out_specs=pl.BlockSpec((bblk, 1), lambda i: (i, 0)),
      out_shape=jax.ShapeDtypeStruct((_B, 1), jnp.float32),
  )(acc, cnt, emb0, W1, b1.reshape(1, -1), W2, b2.reshape(1, -1), Wo,
    bo.reshape(1, -1))


def kernel(text, emb, W1, b1, W2, b2, Wo, bo):
  emb2 = _tc_packT(emb.T).reshape(2 * _S, _D)
  acc, cnt = _sc_pool(text, emb2)
  return _tc_mlp(acc, cnt.reshape(_B, 1), emb[0:1], W1, b1, W2, b2, Wo, bo)


# speedup vs baseline: 1.6582x; 1.0953x over previous
import functools

import jax
import jax.numpy as jnp
from jax import lax
from jax.experimental import pallas as pl
from jax.experimental.pallas import tpu as pltpu
from jax.experimental.pallas import tpu_sc as plsc

_VOCAB = 1000000
_D = 64
_B = 16384
_SEQ = 200

_NC = 2
_NS = 16
_NW = _NC * _NS
_BPW = _B // _NW
_CHUNK = 128
_NCHUNK = _BPW // _CHUNK
_JC = 40
_NJ = _SEQ // _JC


_S = 503808
_VBT = 4096


def _pack_block(a_ref, b_ref, out_ref):
  out_ref[:, 0:_D] = a_ref[...].T
  out_ref[:, _D:2 * _D] = b_ref[...].T


def _tc_packT(embT):
  grid = (_S // _VBT,)
  return pl.pallas_call(
      _pack_block,
      grid=grid,
      in_specs=[
          pl.BlockSpec((_D, _VBT), lambda i: (0, i)),
          pl.BlockSpec(
              (_D, _VBT),
              lambda i: (0, jnp.minimum(i + _S // _VBT,
                                        pl.cdiv(_VOCAB, _VBT) - 1))),
      ],
      out_specs=pl.BlockSpec((_VBT, 2 * _D), lambda i: (i, 0)),
      out_shape=jax.ShapeDtypeStruct((_S, 2 * _D), jnp.float32),
  )(embT, embT)


def _sc_pool(text, emb2):
  mesh = plsc.VectorSubcoreMesh(core_axis_name="c", subcore_axis_name="s")

  @functools.partial(
      pl.kernel,
      out_type=(
          jax.ShapeDtypeStruct((_B, _D), jnp.float32),
          jax.ShapeDtypeStruct((_B,), jnp.float32),
      ),
      mesh=mesh,
      scratch_types=[
          pltpu.VMEM((_BPW, _JC), jnp.int32),
          pltpu.VMEM((_JC, _NCHUNK, _CHUNK), jnp.int32),
          pltpu.VMEM((_JC, _NCHUNK, _CHUNK), jnp.int32),
          pltpu.VMEM((_BPW, _D), jnp.float32),
          pltpu.VMEM((_BPW,), jnp.float32),
          pltpu.SemaphoreType.DMA,
      ],
      compiler_params=pltpu.CompilerParams(
          use_tc_tiling_on_sc=False, needs_layout_passes=False),
  )
  def pool(text_hbm, emb_hbm, out_hbm, cnt_hbm, blk_v, idxTa_v, idxTb_v,
           acc_v, cnt_v, sem):
    wid = lax.axis_index("s") * _NC + lax.axis_index("c")
    base = wid * _BPW
    iota = lax.broadcasted_iota(jnp.int32, (16,), 0)
    idx_bufs = (idxTa_v, idxTb_v)

    zero16 = jnp.zeros((16,), jnp.float32)
    for z in range(_BPW // 16):
      cnt_v[pl.ds(z * 16, 16)] = zero16

    @pl.loop(0, _BPW)
    def _(r):
      for k in range(_D // 16):
        acc_v[r, pl.ds(k * 16, 16)] = zero16

    def load_and_transpose(h, idxT_v):
      pltpu.sync_copy(
          text_hbm.at[pl.ds(base, _BPW), pl.ds(h * _JC, _JC)], blk_v)

      @pl.loop(0, _JC)
      def _(j):
        jcol = jnp.full((16,), j, jnp.int32)
        for c in range(_NCHUNK):
          for lb in range(_CHUNK // 16):
            row0 = c * _CHUNK + lb * 16
            vals = plsc.load_gather(blk_v, [iota + row0, jcol])
            vals2 = vals + vals
            idxT_v[j, c, pl.ds(lb * 16, 16)] = jnp.where(
                vals < _S, vals2, vals2 - (2 * _S - 1))
            cnt_v[pl.ds(row0, 16)] = cnt_v[pl.ds(row0, 16)] + jnp.where(
                vals == 0, 1.0, 0.0)

    load_and_transpose(0, idxTa_v)
    for h in range(_NJ):
      idxT_v = idx_bufs[h % 2]

      @pl.loop(0, _JC)
      def _(j):
        for c in range(_NCHUNK):
          pltpu.async_copy(emb_hbm.at[idxT_v.at[j, c]],
                           acc_v.at[pl.ds(c * _CHUNK, _CHUNK)], sem, add=True)

      if h + 1 < _NJ:
        load_and_transpose(h + 1, idx_bufs[(h + 1) % 2])

      @pl.loop(0, _JC * _NCHUNK)
      def _(i):
        pltpu.make_async_copy(emb_hbm.at[idxT_v.at[0, 0]],
                              acc_v.at[pl.ds(0, _CHUNK)], sem).wait()

    pltpu.sync_copy(acc_v, out_hbm.at[pl.ds(base, _BPW)])
    pltpu.sync_copy(cnt_v, cnt_hbm.at[pl.ds(base, _BPW)])

  return pool(text, emb2)


def _mlp_block(acc_ref, cnt_ref, emb0_ref, w1_ref, b1_ref, w2_ref, b2_ref,
               wo_ref, bo_ref, out_ref):
  pooled = (acc_ref[...] - cnt_ref[...] * emb0_ref[...]) * (1.0 / _SEQ)
  h = jnp.dot(pooled, w1_ref[...], preferred_element_type=jnp.float32)
  h = jnp.maximum(h + b1_ref[...], 0.0)
  h = jnp.dot(h, w2_ref[...], preferred_element_type=jnp.float32)
  h = jnp.maximum(h + b2_ref[...], 0.0)
  out_ref[...] = (
      jnp.dot(h, wo_ref[...], preferred_element_type=jnp.float32)
      + bo_ref[...])


def _tc_mlp(acc, cnt, emb0, W1, b1, W2, b2, Wo, bo):
  bblk = 2048
  grid = (_B // bblk,)
  full = lambda shape: pl.BlockSpec(shape, lambda i: (0, 0))
  return pl.pallas_call(
      _mlp_block,
      grid=grid,
      in_specs=[
          pl.BlockSpec((bblk, _D), lambda i: (i, 0)),
          pl.BlockSpec((bblk, 1), lambda i: (i, 0)),
          full((1, _D)),
          full(W1.shape),
          full((1, 256)),
          full(W2.shape),
          full((1, 128)),
          full(Wo.shape),
          full((1, 1)),
      ],
      out_specs=pl.BlockSpec((bblk, 1), lambda i: (i, 0)),
      out_shape=jax.ShapeDtypeStruct((_B, 1), jnp.float32),
  )(acc, cnt, emb0, W1, b1.reshape(1, -1), W2, b2.reshape(1, -1), Wo,
    bo.reshape(1, -1))


def kernel(text, emb, W1, b1, W2, b2, Wo, bo):
  emb2 = _tc_packT(emb.T).reshape(2 * _S, _D)
  acc, cnt = _sc_pool(text, emb2)
  return _tc_mlp(acc, cnt.reshape(_B, 1), emb[0:1], W1, b1, W2, b2, Wo, bo)


# speedup vs baseline: 1.7461x; 1.0531x over previous
import functools

import jax
import jax.numpy as jnp
from jax import lax
from jax.experimental import pallas as pl
from jax.experimental.pallas import tpu as pltpu
from jax.experimental.pallas import tpu_sc as plsc

_VOCAB = 1000000
_D = 64
_B = 16384
_SEQ = 200

_NC = 2
_NS = 16
_NW = _NC * _NS
_BPW = _B // _NW
_CHUNK = 128
_NCHUNK = _BPW // _CHUNK
_JC = 40
_NJ = _SEQ // _JC


_S = 507904
_VBT = 8192


def _pack_block(a_ref, b_ref, out_ref):
  out_ref[:, 0:_D] = a_ref[...].T
  out_ref[:, _D:2 * _D] = b_ref[...].T


def _tc_packT(embT):
  grid = (_S // _VBT,)
  return pl.pallas_call(
      _pack_block,
      grid=grid,
      in_specs=[
          pl.BlockSpec((_D, _VBT), lambda i: (0, i)),
          pl.BlockSpec(
              (_D, _VBT),
              lambda i: (0, jnp.minimum(i + _S // _VBT,
                                        pl.cdiv(_VOCAB, _VBT) - 1))),
      ],
      out_specs=pl.BlockSpec((_VBT, 2 * _D), lambda i: (i, 0)),
      out_shape=jax.ShapeDtypeStruct((_S, 2 * _D), jnp.float32),
  )(embT, embT)


def _sc_pool(text, emb2):
  mesh = plsc.VectorSubcoreMesh(core_axis_name="c", subcore_axis_name="s")

  @functools.partial(
      pl.kernel,
      out_type=(
          jax.ShapeDtypeStruct((_B, _D), jnp.float32),
          jax.ShapeDtypeStruct((_B,), jnp.float32),
      ),
      mesh=mesh,
      scratch_types=[
          pltpu.VMEM((_BPW, _JC), jnp.int32),
          pltpu.VMEM((_JC, _NCHUNK, _CHUNK), jnp.int32),
          pltpu.VMEM((_JC, _NCHUNK, _CHUNK), jnp.int32),
          pltpu.VMEM((_BPW, _D), jnp.float32),
          pltpu.VMEM((_BPW,), jnp.float32),
          pltpu.SemaphoreType.DMA,
      ],
      compiler_params=pltpu.CompilerParams(
          use_tc_tiling_on_sc=False, needs_layout_passes=False),
  )
  def pool(text_hbm, emb_hbm, out_hbm, cnt_hbm, blk_v, idxTa_v, idxTb_v,
           acc_v, cnt_v, sem):
    wid = lax.axis_index("s") * _NC + lax.axis_index("c")
    base = wid * _BPW
    iota = lax.broadcasted_iota(jnp.int32, (16,), 0)
    idx_bufs = (idxTa_v, idxTb_v)

    zero16 = jnp.zeros((16,), jnp.float32)
    for z in range(_BPW // 16):
      cnt_v[pl.ds(z * 16, 16)] = zero16

    @pl.loop(0, _BPW)
    def _(r):
      for k in range(_D // 16):
        acc_v[r, pl.ds(k * 16, 16)] = zero16

    def load_and_transpose(h, idxT_v):
      pltpu.sync_copy(
          text_hbm.at[pl.ds(base, _BPW), pl.ds(h * _JC, _JC)], blk_v)

      @pl.loop(0, _JC)
      def _(j):
        jcol = jnp.full((16,), j, jnp.int32)
        for c in range(_NCHUNK):
          for lb in range(_CHUNK // 16):
            row0 = c * _CHUNK + lb * 16
            vals = plsc.load_gather(blk_v, [iota + row0, jcol])
            vals2 = vals + vals
            idxT_v[j, c, pl.ds(lb * 16, 16)] = jnp.where(
                vals < _S, vals2, vals2 - (2 * _S - 1))
            cnt_v[pl.ds(row0, 16)] = cnt_v[pl.ds(row0, 16)] + jnp.where(
                vals == 0, 1.0, 0.0)

    load_and_transpose(0, idxTa_v)
    for h in range(_NJ):
      idxT_v = idx_bufs[h % 2]

      @pl.loop(0, _JC)
      def _(j):
        for c in range(_NCHUNK):
          pltpu.async_copy(emb_hbm.at[idxT_v.at[j, c]],
                           acc_v.at[pl.ds(c * _CHUNK, _CHUNK)], sem, add=True)

      if h + 1 < _NJ:
        load_and_transpose(h + 1, idx_bufs[(h + 1) % 2])

      @pl.loop(0, _JC * _NCHUNK)
      def _(i):
        pltpu.make_async_copy(emb_hbm.at[idxT_v.at[0, 0]],
                              acc_v.at[pl.ds(0, _CHUNK)], sem).wait()

    pltpu.sync_copy(acc_v, out_hbm.at[pl.ds(base, _BPW)])
    pltpu.sync_copy(cnt_v, cnt_hbm.at[pl.ds(base, _BPW)])

  return pool(text, emb2)


def _mlp_block(acc_ref, cnt_ref, emb0_ref, w1_ref, b1_ref, w2_ref, b2_ref,
               wo_ref, bo_ref, out_ref):
  pooled = (acc_ref[...] - cnt_ref[...] * emb0_ref[...]) * (1.0 / _SEQ)
  h = jnp.dot(pooled, w1_ref[...], preferred_element_type=jnp.float32)
  h = jnp.maximum(h + b1_ref[...], 0.0)
  h = jnp.dot(h, w2_ref[...], preferred_element_type=jnp.float32)
  h = jnp.maximum(h + b2_ref[...], 0.0)
  out_ref[...] = (
      jnp.dot(h, wo_ref[...], preferred_element_type=jnp.float32)
      + bo_ref[...])


def _tc_mlp(acc, cnt, emb0, W1, b1, W2, b2, Wo, bo):
  bblk = 2048
  grid = (_B // bblk,)
  full = lambda shape: pl.BlockSpec(shape, lambda i: (0, 0))
  return pl.pallas_call(
      _mlp_block,
      grid=grid,
      in_specs=[
          pl.BlockSpec((bblk, _D), lambda i: (i, 0)),
          pl.BlockSpec((bblk, 1), lambda i: (i, 0)),
          full((1, _D)),
          full(W1.shape),
          full((1, 256)),
          full(W2.shape),
          full((1, 128)),
          full(Wo.shape),
          full((1, 1)),
      ],
      out_specs=pl.BlockSpec((bblk, 1), lambda i: (i, 0)),
      out_shape=jax.ShapeDtypeStruct((_B, 1), jnp.float32),
  )(acc, cnt, emb0, W1, b1.reshape(1, -1), W2, b2.reshape(1, -1), Wo,
    bo.reshape(1, -1))


def kernel(text, emb, W1, b1, W2, b2, Wo, bo):
  emb2 = _tc_packT(emb.T).reshape(2 * _S, _D)
  acc, cnt = _sc_pool(text, emb2)
  return _tc_mlp(acc, cnt.reshape(_B, 1), emb[0:1], W1, b1, W2, b2, Wo, bo)


# speedup vs baseline: 1.7871x; 1.0235x over previous
import functools

import jax
import jax.numpy as jnp
from jax import lax
from jax.experimental import pallas as pl
from jax.experimental.pallas import tpu as pltpu
from jax.experimental.pallas import tpu_sc as plsc

_VOCAB = 1000000
_D = 64
_B = 16384
_SEQ = 200

_NC = 2
_NS = 16
_NW = _NC * _NS
_BPW = _B // _NW
_CHUNK = 128
_NCHUNK = _BPW // _CHUNK
_JC = 40
_NJ = _SEQ // _JC


_S = 507904
_VBT = 16384


def _pack_block(a_ref, b_ref, out_ref):
  out_ref[:, 0:_D] = a_ref[...].T
  out_ref[:, _D:2 * _D] = b_ref[...].T


def _tc_packT(embT):
  grid = (_S // _VBT,)
  return pl.pallas_call(
      _pack_block,
      grid=grid,
      in_specs=[
          pl.BlockSpec((_D, _VBT), lambda i: (0, i)),
          pl.BlockSpec(
              (_D, _VBT),
              lambda i: (0, jnp.minimum(i + _S // _VBT,
                                        pl.cdiv(_VOCAB, _VBT) - 1))),
      ],
      out_specs=pl.BlockSpec((_VBT, 2 * _D), lambda i: (i, 0)),
      out_shape=jax.ShapeDtypeStruct((_S, 2 * _D), jnp.float32),
  )(embT, embT)


def _sc_pool(text, emb2):
  mesh = plsc.VectorSubcoreMesh(core_axis_name="c", subcore_axis_name="s")

  @functools.partial(
      pl.kernel,
      out_type=(
          jax.ShapeDtypeStruct((_B, _D), jnp.float32),
          jax.ShapeDtypeStruct((_B,), jnp.float32),
      ),
      mesh=mesh,
      scratch_types=[
          pltpu.VMEM((_BPW, _JC), jnp.int32),
          pltpu.VMEM((_JC, _NCHUNK, _CHUNK), jnp.int32),
          pltpu.VMEM((_JC, _NCHUNK, _CHUNK), jnp.int32),
          pltpu.VMEM((_BPW, _D), jnp.float32),
          pltpu.VMEM((_BPW,), jnp.float32),
          pltpu.SemaphoreType.DMA,
      ],
      compiler_params=pltpu.CompilerParams(
          use_tc_tiling_on_sc=False, needs_layout_passes=False),
  )
  def pool(text_hbm, emb_hbm, out_hbm, cnt_hbm, blk_v, idxTa_v, idxTb_v,
           acc_v, cnt_v, sem):
    wid = lax.axis_index("s") * _NC + lax.axis_index("c")
    base = wid * _BPW
    iota = lax.broadcasted_iota(jnp.int32, (16,), 0)
    idx_bufs = (idxTa_v, idxTb_v)

    zero16 = jnp.zeros((16,), jnp.float32)
    for z in range(_BPW // 16):
      cnt_v[pl.ds(z * 16, 16)] = zero16

    @pl.loop(0, _BPW)
    def _(r):
      for k in range(_D // 16):
        acc_v[r, pl.ds(k * 16, 16)] = zero16

    def load_and_transpose(h, idxT_v):
      pltpu.sync_copy(
          text_hbm.at[pl.ds(base, _BPW), pl.ds(h * _JC, _JC)], blk_v)

      @pl.loop(0, _JC)
      def _(j):
        jcol = jnp.full((16,), j, jnp.int32)
        for c in range(_NCHUNK):
          for lb in range(_CHUNK // 16):
            row0 = c * _CHUNK + lb * 16
            vals = plsc.load_gather(blk_v, [iota + row0, jcol])
            vals2 = vals + vals
            idxT_v[j, c, pl.ds(lb * 16, 16)] = jnp.where(
                vals < _S, vals2, vals2 - (2 * _S - 1))
            cnt_v[pl.ds(row0, 16)] = cnt_v[pl.ds(row0, 16)] + jnp.where(
                vals == 0, 1.0, 0.0)

    load_and_transpose(0, idxTa_v)
    for h in range(_NJ):
      idxT_v = idx_bufs[h % 2]

      @pl.loop(0, _JC)
      def _(j):
        for c in range(_NCHUNK):
          pltpu.async_copy(emb_hbm.at[idxT_v.at[j, c]],
                           acc_v.at[pl.ds(c * _CHUNK, _CHUNK)], sem, add=True)

      if h + 1 < _NJ:
        load_and_transpose(h + 1, idx_bufs[(h + 1) % 2])

      @pl.loop(0, _JC * _NCHUNK)
      def _(i):
        pltpu.make_async_copy(emb_hbm.at[idxT_v.at[0, 0]],
                              acc_v.at[pl.ds(0, _CHUNK)], sem).wait()

    pltpu.sync_copy(acc_v, out_hbm.at[pl.ds(base, _BPW)])
    pltpu.sync_copy(cnt_v, cnt_hbm.at[pl.ds(base, _BPW)])

  return pool(text, emb2)


def _mlp_block(acc_ref, cnt_ref, emb0_ref, w1_ref, b1_ref, w2_ref, b2_ref,
               wo_ref, bo_ref, out_ref):
  pooled = (acc_ref[...] - cnt_ref[...] * emb0_ref[...]) * (1.0 / _SEQ)
  h = jnp.dot(pooled, w1_ref[...], preferred_element_type=jnp.float32)
  h = jnp.maximum(h + b1_ref[...], 0.0)
  h = jnp.dot(h, w2_ref[...], preferred_element_type=jnp.float32)
  h = jnp.maximum(h + b2_ref[...], 0.0)
  out_ref[...] = (
      jnp.dot(h, wo_ref[...], preferred_element_type=jnp.float32)
      + bo_ref[...])


def _tc_mlp(acc, cnt, emb0, W1, b1, W2, b2, Wo, bo):
  bblk = 2048
  grid = (_B // bblk,)
  full = lambda shape: pl.BlockSpec(shape, lambda i: (0, 0))
  return pl.pallas_call(
      _mlp_block,
      grid=grid,
      in_specs=[
          pl.BlockSpec((bblk, _D), lambda i: (i, 0)),
          pl.BlockSpec((bblk, 1), lambda i: (i, 0)),
          full((1, _D)),
          full(W1.shape),
          full((1, 256)),
          full(W2.shape),
          full((1, 128)),
          full(Wo.shape),
          full((1, 1)),
      ],
      out_specs=pl.BlockSpec((bblk, 1), lambda i: (i, 0)),
      out_shape=jax.ShapeDtypeStruct((_B, 1), jnp.float32),
  )(acc, cnt, emb0, W1, b1.reshape(1, -1), W2, b2.reshape(1, -1), Wo,
    bo.reshape(1, -1))


def kernel(text, emb, W1, b1, W2, b2, Wo, bo):
  emb2 = _tc_packT(emb.T).reshape(2 * _S, _D)
  acc, cnt = _sc_pool(text, emb2)
  return _tc_mlp(acc, cnt.reshape(_B, 1), emb[0:1], W1, b1, W2, b2, Wo, bo)
